# Initial kernel scaffold; baseline (speedup 1.0000x reference)
import jax
import jax.numpy as jnp
from jax.experimental import pallas as pl


def kernel(is_object, position, output_hs, gt_boxes, obj_idx, gt_obj_ids):
    raise NotImplementedError("write your pallas kernel here")


# speedup vs baseline: 6557.1755x; 6557.1755x over previous
import jax
import jax.numpy as jnp
from jax import lax
from jax.experimental import pallas as pl
from jax.experimental.pallas import tpu as pltpu

_DET_THRESH = 0.5


def _match_kernel(obj_ref, oi_ref, pxy_ref, gtt_ref, gidrow_ref, gid_smem,
                  dist_ref, score_ref, gtidx_ref, objix_ref, dbg_ref,
                  rowpen_ref, newj_ref, colmin_ref, colarg_ref):
    P = obj_ref.shape[0]
    M = gtt_ref.shape[1]
    INF = jnp.float32(jnp.inf)
    i32 = jnp.int32
    BIG = jnp.int32(2**30)

    x = pxy_ref[:, 0:1]
    y = pxy_ref[:, 1:2]
    gx = gtt_ref[0:1, :]
    gy = gtt_ref[1:2, :]
    dist = (x - gx) ** 2 + (y - gy) ** 2
    dist_ref[...] = dist

    eq = oi_ref[...] == gidrow_ref[...]
    j_iota = lax.broadcasted_iota(i32, (P, M), 1)
    firstj = jnp.min(jnp.where(eq, j_iota, M), axis=1, keepdims=True)
    has_pr = firstj < M
    a_gt0 = jnp.max(eq.astype(i32), axis=0, keepdims=True) > 0

    rowpen_ref[...] = jnp.where(has_pr, INF, jnp.float32(0.0))
    newj_ref[...] = jnp.full((P, 1), -1, i32)
    objix_ref[...] = oi_ref[...]

    i_iota = lax.broadcasted_iota(i32, (P, M), 0)
    d0 = dist + rowpen_ref[...]
    cm0 = jnp.min(d0, axis=0, keepdims=True)
    ca0 = jnp.min(jnp.where(d0 == cm0, i_iota, BIG), axis=0, keepdims=True)
    colmin_ref[...] = jnp.where(a_gt0, INF, cm0)
    colarg_ref[...] = ca0.astype(i32)

    jiota_row = lax.broadcasted_iota(i32, (1, M), 1)

    def body(_, carry):
        colmin = colmin_ref[...]
        colarg = colarg_ref[...]
        m = jnp.min(colmin)
        cand = jnp.where(colmin == m, colarg * M + jiota_row, BIG)
        k = jnp.min(cand)
        i = k // M
        j = k - i * M

        @pl.when(m < INF)
        def _():
            newj_ref[pl.ds(i, 1), :] = jnp.full((1, 1), j, i32)
            objix_ref[pl.ds(i, 1), :] = jnp.full((1, 1), gid_smem[j], i32)
            rowpen_ref[pl.ds(i, 1), :] = jnp.full((1, 1), INF)
            colmin2 = jnp.where(jiota_row == j, INF, colmin)
            colmin_ref[...] = colmin2
            stale = (colarg == i) & (colmin2 < INF)

            @pl.when(jnp.max(stale.astype(i32)) > 0)
            def _():
                d = dist_ref[...] + rowpen_ref[...]
                nm = jnp.min(d, axis=0, keepdims=True)
                na = jnp.min(jnp.where(d == nm, i_iota, BIG),
                             axis=0, keepdims=True)
                colmin_ref[...] = jnp.where(colmin2 == INF, INF, nm)
                colarg_ref[...] = na.astype(i32)

        return carry

    lax.fori_loop(jnp.int32(0), jnp.int32(M), body, jnp.int32(0))

    newj = newj_ref[...]
    assigned = newj >= 0
    gtidx_ref[...] = jnp.where(assigned, newj,
                               jnp.where(has_pr, firstj, jnp.int32(-1))).astype(i32)
    score_ref[...] = jax.nn.sigmoid(obj_ref[...])
    dbg_ref[...] = (jnp.where(assigned, jnp.int32(3),
                              jnp.where(has_pr, jnp.int32(2), jnp.int32(0)))
                    + jnp.where(obj_ref[...] > _DET_THRESH,
                                jnp.int32(10), jnp.int32(0))).astype(i32)


def kernel(is_object, position, output_hs, gt_boxes, obj_idx, gt_obj_ids):
    P = obj_idx.shape[0]
    M = gt_obj_ids.shape[0]

    obj = is_object[-1, 0, :, :]
    oi = obj_idx.astype(jnp.int32).reshape(P, 1)
    pxy = position[-1, 0, :, :2]
    gtt = gt_boxes[:, :2].T
    gid32 = gt_obj_ids.astype(jnp.int32)
    gidrow = gid32.reshape(1, M)

    dist, score, gtidx, objix, dbg = pl.pallas_call(
        _match_kernel,
        out_shape=[
            jax.ShapeDtypeStruct((P, M), jnp.float32),
            jax.ShapeDtypeStruct((P, 1), jnp.float32),
            jax.ShapeDtypeStruct((P, 1), jnp.int32),
            jax.ShapeDtypeStruct((P, 1), jnp.int32),
            jax.ShapeDtypeStruct((P, 1), jnp.int32),
        ],
        in_specs=[
            pl.BlockSpec((P, 1), lambda: (0, 0)),
            pl.BlockSpec((P, 1), lambda: (0, 0)),
            pl.BlockSpec((P, 2), lambda: (0, 0)),
            pl.BlockSpec((2, M), lambda: (0, 0)),
            pl.BlockSpec((1, M), lambda: (0, 0)),
            pl.BlockSpec(memory_space=pltpu.SMEM),
        ],
        scratch_shapes=[
            pltpu.VMEM((P, 1), jnp.float32),
            pltpu.VMEM((P, 1), jnp.int32),
            pltpu.VMEM((1, M), jnp.float32),
            pltpu.VMEM((1, M), jnp.int32),
        ],
    )(obj, oi, pxy, gtt, gidrow, gid32)

    score = score.reshape(P)
    gt_idx = gtidx.reshape(P).astype(jnp.int64)
    obj_ix = objix.reshape(P).astype(jnp.int64)
    dbg = dbg.reshape(P)
    q_ref = position[-1, 0]
    q_emb = output_hs[-1, 0]
    return (score, dist, dbg, gt_idx, obj_ix, q_ref, q_emb)
